# baseline (speedup 1.0000x reference)
import functools

import jax
import jax.numpy as jnp
from jax import lax
from jax.experimental import pallas as pl
from jax.experimental.pallas import tpu as pltpu
from jax.experimental.pallas import tpu_sc as plsc

N = 10000
F = 128
T = 8
U = 16
NPAD = 10240
SCRAP = 10016
NSUB = 16
NCORES = 2
CHUNK = 128
RPT = NPAD // NSUB
NB = 400
GRID_N = N // NB


@functools.lru_cache(maxsize=1)
def _mesh():
    return plsc.VectorSubcoreMesh(
        core_axis_name="c", subcore_axis_name="s",
        num_cores=NCORES, num_subcores=NSUB)


_PREC = lax.Precision.HIGHEST


def _sc_deg_body(dst_hbm, ones_hbm, zeros_hbm, out_hbm, idx_v, ones_v, acc, sem):
    c = lax.axis_index("c")
    s = lax.axis_index("s")
    nchunks = dst_hbm.shape[1]
    half = (nchunks + 1) // 2
    pltpu.sync_copy(ones_hbm, ones_v)
    pltpu.sync_copy(zeros_hbm, acc.at[pl.ds(s * RPT, RPT)])
    plsc.subcore_barrier()

    def step(j, carry):
        pltpu.sync_copy(dst_hbm.at[s].at[j], idx_v)
        pltpu.sync_copy(ones_v, acc.at[idx_v], add=True)
        return carry

    lax.fori_loop(c * half, half + (nchunks - half) * c, step, 0)
    plsc.subcore_barrier()
    pltpu.sync_copy(acc.at[pl.ds(s * RPT, RPT)],
                    out_hbm.at[c].at[pl.ds(s * RPT, RPT)])


def _sc_agg_body(y_hbm, src_hbm, dst_hbm, zeros_hbm, out_hbm,
                 idxs_v, idxd_v, gbuf, acc, sem):
    c = lax.axis_index("c")
    s = lax.axis_index("s")
    nchunks = src_hbm.shape[1]
    pltpu.sync_copy(zeros_hbm, acc.at[pl.ds(s * RPT, RPT)])
    plsc.subcore_barrier()
    for tl in range(T // NCORES):
        t = c * (T // NCORES) + tl

        def step(j, carry):
            pltpu.sync_copy(src_hbm.at[s].at[j], idxs_v)
            pltpu.sync_copy(dst_hbm.at[s].at[j], idxd_v)
            pltpu.async_copy(y_hbm.at[t].at[idxs_v], gbuf, sem).wait()
            pltpu.sync_copy(gbuf, acc.at[idxd_v], add=True)
            return carry

        lax.fori_loop(0, nchunks, step, 0)
        plsc.subcore_barrier()
        pltpu.sync_copy(acc.at[pl.ds(s * RPT, RPT)],
                        out_hbm.at[t].at[pl.ds(s * RPT, RPT)])
        if tl < T // NCORES - 1:
            pltpu.sync_copy(zeros_hbm, acc.at[pl.ds(s * RPT, RPT)])
            plsc.subcore_barrier()


def _sc_degree(dst_tiles, ones16, zeros16):
    call = pl.kernel(
        _sc_deg_body,
        out_type=jax.ShapeDtypeStruct((NCORES, NPAD, F), jnp.float32),
        mesh=_mesh(),
        scratch_types=[
            pltpu.VMEM((CHUNK,), jnp.int32),
            pltpu.VMEM((CHUNK, F), jnp.float32),
            pltpu.VMEM_SHARED((NPAD, F), jnp.float32),
            pltpu.SemaphoreType.DMA,
        ],
    )
    return call(dst_tiles, ones16, zeros16)


def _sc_aggregate(y, src_tiles, dst_tiles, zerosf):
    call = pl.kernel(
        _sc_agg_body,
        out_type=jax.ShapeDtypeStruct((T, NPAD, F), jnp.float32),
        mesh=_mesh(),
        scratch_types=[
            pltpu.VMEM((CHUNK,), jnp.int32),
            pltpu.VMEM((CHUNK,), jnp.int32),
            pltpu.VMEM((CHUNK, F), jnp.float32),
            pltpu.VMEM_SHARED((NPAD, F), jnp.float32),
            pltpu.SemaphoreType.DMA,
        ],
    )
    return call(y, src_tiles, dst_tiles, zerosf)


def _rsqrt_deg(d_ref):
    return lax.rsqrt(d_ref[0, :, 0:1] + d_ref[1, :, 0:1] + 1.0)


def _scale_body(g_ref, d_ref, o_ref):
    o_ref[0] = g_ref[0] * _rsqrt_deg(d_ref)


def _layer_body(acc_ref, y_ref, d_ref, w_ref, b_ref, o_ref):
    r = _rsqrt_deg(d_ref)
    sx = (acc_ref[0] + y_ref[0]) * r
    h = jnp.maximum(
        jnp.dot(sx, w_ref[...], preferred_element_type=jnp.float32,
                precision=_PREC) + b_ref[...][None, :], 0.0)
    o_ref[0] = h * r


def _tail_body(acc_ref, y_ref, d_ref, w2_ref, b2_ref, wd1_ref, bd1_ref,
               wd2_ref, bd2_ref, wi_ref, bl_ref, o_ref):
    r = _rsqrt_deg(d_ref)
    sx = (acc_ref[0] + y_ref[0]) * r
    h2 = jnp.maximum(
        jnp.dot(sx, w2_ref[...], preferred_element_type=jnp.float32,
                precision=_PREC) + b2_ref[...][None, :], 0.0)
    x3 = jnp.dot(h2, wd1_ref[...], preferred_element_type=jnp.float32,
                 precision=_PREC) + bd1_ref[...][None, :]
    x4 = jnp.dot(x3, wd2_ref[...], preferred_element_type=jnp.float32,
                 precision=_PREC) + bd2_ref[...][None, :]
    xw = jnp.dot(x4, wi_ref[...], preferred_element_type=jnp.float32,
                 precision=_PREC) + bl_ref[...][None, :]
    o_ref[0] = xw


def _lstm_body(x_ref, wh_ref, o_ref):
    wh = wh_ref[...]

    def step(i, hc):
        h, c = hc
        z = x_ref[i] + jnp.dot(h, wh, preferred_element_type=jnp.float32,
                               precision=_PREC)
        gi = jax.nn.sigmoid(z[:, 0:U])
        gf = jax.nn.sigmoid(z[:, U:2 * U])
        gg = jnp.tanh(z[:, 2 * U:3 * U])
        go = jax.nn.sigmoid(z[:, 3 * U:4 * U])
        c2 = gf * c + gi * gg
        h2 = go * jnp.tanh(c2)
        return (h2, c2)

    init = (jnp.zeros((T, U), jnp.float32), jnp.zeros((T, U), jnp.float32))
    h, _ = lax.fori_loop(0, x_ref.shape[0], step, init)
    o_ref[...] = h


def _final_body(flat_ref, w_ref, b_ref, o_ref):
    o_ref[...] = jnp.dot(flat_ref[...], w_ref[...],
                         preferred_element_type=jnp.float32,
                         precision=_PREC) + b_ref[...]


def _tc_scale(graphs, degp):
    return pl.pallas_call(
        _scale_body,
        grid=(T, GRID_N),
        in_specs=[
            pl.BlockSpec((1, NB, F), lambda t, n: (t, n, 0)),
            pl.BlockSpec((NCORES, NB, 128), lambda t, n: (0, n, 0)),
        ],
        out_specs=pl.BlockSpec((1, NB, F), lambda t, n: (t, n, 0)),
        out_shape=jax.ShapeDtypeStruct((T, N, F), jnp.float32),
    )(graphs, degp)


def _tc_layer(acc, y, degp, w, b):
    return pl.pallas_call(
        _layer_body,
        grid=(T, GRID_N),
        in_specs=[
            pl.BlockSpec((1, NB, F), lambda t, n: (t, n, 0)),
            pl.BlockSpec((1, NB, F), lambda t, n: (t, n, 0)),
            pl.BlockSpec((NCORES, NB, 128), lambda t, n: (0, n, 0)),
            pl.BlockSpec((F, F), lambda t, n: (0, 0)),
            pl.BlockSpec((F,), lambda t, n: (0,)),
        ],
        out_specs=pl.BlockSpec((1, NB, F), lambda t, n: (t, n, 0)),
        out_shape=jax.ShapeDtypeStruct((T, N, F), jnp.float32),
    )(acc, y, degp, w, b)


def _tc_tail(acc, y, degp, W2, b2, Wd1, bd1, Wd2, bd2, Wi, bl):
    wspec = pl.BlockSpec((F, F), lambda t, n: (0, 0))
    bspec = pl.BlockSpec((F,), lambda t, n: (0,))
    return pl.pallas_call(
        _tail_body,
        grid=(T, GRID_N),
        in_specs=[
            pl.BlockSpec((1, NB, F), lambda t, n: (t, n, 0)),
            pl.BlockSpec((1, NB, F), lambda t, n: (t, n, 0)),
            pl.BlockSpec((NCORES, NB, 128), lambda t, n: (0, n, 0)),
            wspec, bspec, wspec, bspec, wspec, bspec,
            pl.BlockSpec((F, 4 * U), lambda t, n: (0, 0)),
            pl.BlockSpec((4 * U,), lambda t, n: (0,)),
        ],
        out_specs=pl.BlockSpec((1, NB, 4 * U), lambda t, n: (t, n, 0)),
        out_shape=jax.ShapeDtypeStruct((T, N, 4 * U), jnp.float32),
    )(acc, y, degp, W2, b2, Wd1, bd1, Wd2, bd2, Wi, bl)


def _tc_lstm(xw, Wh):
    return pl.pallas_call(
        _lstm_body,
        out_shape=jax.ShapeDtypeStruct((T, U), jnp.float32),
    )(xw, Wh)


def _tc_final(flat, Wd3, bd3):
    CB = 6400
    return pl.pallas_call(
        _final_body,
        grid=(N * U // CB,),
        in_specs=[
            pl.BlockSpec((1, T * U), lambda n: (0, 0)),
            pl.BlockSpec((T * U, CB), lambda n: (0, n)),
            pl.BlockSpec((1, CB), lambda n: (0, n)),
        ],
        out_specs=pl.BlockSpec((1, CB), lambda n: (0, n)),
        out_shape=jax.ShapeDtypeStruct((1, N * U), jnp.float32),
    )(flat, Wd3, bd3)


def kernel(graphs, edge_index, edge_attr, W1, b1, W2, b2, Wd1, bd1, Wd2, bd2,
           Wi, Wh, bl, Wd3, bd3):
    src = edge_index[0]
    dst = edge_index[1]
    E = src.shape[0]
    per_tile = -(-(E // NSUB) // CHUNK) * CHUNK
    tot = per_tile * NSUB
    pad = tot - E
    src_tiles = jnp.concatenate(
        [src, jnp.zeros((pad,), jnp.int32)]).reshape(NSUB, -1, CHUNK)
    dst_tiles = jnp.concatenate(
        [dst, jnp.full((pad,), SCRAP, jnp.int32)]).reshape(NSUB, -1, CHUNK)
    ones16 = jnp.ones((CHUNK, F), jnp.float32)
    zeros16 = jnp.zeros((RPT, F), jnp.float32)
    zerosf = jnp.zeros((RPT, F), jnp.float32)

    degp = _sc_degree(dst_tiles, ones16, zeros16)
    y0 = _tc_scale(graphs, degp)
    acc1 = _sc_aggregate(y0, src_tiles, dst_tiles, zerosf)
    y1 = _tc_layer(acc1, y0, degp, W1, b1)
    acc2 = _sc_aggregate(y1, src_tiles, dst_tiles, zerosf)
    xw = _tc_tail(acc2, y1, degp, W2, b2, Wd1, bd1, Wd2, bd2, Wi, bl)
    h_last = _tc_lstm(jnp.swapaxes(xw, 0, 1), Wh)
    out = _tc_final(h_last.reshape(1, T * U), Wd3, bd3.reshape(1, N * U))
    return out.reshape(N, U)


# speedup vs baseline: 1.9796x; 1.9796x over previous
import functools

import jax
import jax.numpy as jnp
from jax import lax
from jax.experimental import pallas as pl
from jax.experimental.pallas import tpu as pltpu
from jax.experimental.pallas import tpu_sc as plsc

N = 10000
F = 128
T = 8
U = 16
NPAD = 10240
SCRAP = 10016
NSUB = 16
NCORES = 2
CHUNK = 120
NSLOT = 3
RPT = NPAD // NSUB
NB = 400
GRID_N = N // NB


@functools.lru_cache(maxsize=1)
def _mesh():
    return plsc.VectorSubcoreMesh(
        core_axis_name="c", subcore_axis_name="s",
        num_cores=NCORES, num_subcores=NSUB)


_PREC = lax.Precision.HIGHEST


def _sc_deg_body(dst_hbm, ones_hbm, zeros_hbm, out_hbm, idx_v, ones_v, acc, sem):
    c = lax.axis_index("c")
    s = lax.axis_index("s")
    nchunks = dst_hbm.shape[1]
    half = (nchunks + 1) // 2
    pltpu.sync_copy(ones_hbm, ones_v)
    pltpu.sync_copy(zeros_hbm, acc.at[pl.ds(s * RPT, RPT)])
    plsc.subcore_barrier()

    def step(j, carry):
        pltpu.sync_copy(dst_hbm.at[s].at[j], idx_v)
        pltpu.sync_copy(ones_v, acc.at[idx_v.at[1]], add=True)
        return carry

    lax.fori_loop(c * half, half + (nchunks - half) * c, step, 0)
    plsc.subcore_barrier()
    pltpu.sync_copy(acc.at[pl.ds(s * RPT, RPT)],
                    out_hbm.at[c].at[pl.ds(s * RPT, RPT)])


def _sc_agg_body(y_hbm, idx_hbm, zeros_hbm, out_hbm,
                 idx_v, gbuf, acc, semi, semg):
    c = lax.axis_index("c")
    s = lax.axis_index("s")
    nchunks = idx_hbm.shape[1]
    pltpu.sync_copy(zeros_hbm, acc.at[pl.ds(s * RPT, RPT)])
    plsc.subcore_barrier()
    for tl in range(T // NCORES):
        t = c * (T // NCORES) + tl
        pltpu.async_copy(idx_hbm.at[s].at[0], idx_v.at[0], semi.at[0])
        pltpu.async_copy(idx_hbm.at[s].at[1], idx_v.at[1], semi.at[1])
        pltpu.make_async_copy(idx_hbm.at[s].at[0], idx_v.at[0],
                              semi.at[0]).wait()
        pltpu.async_copy(y_hbm.at[t].at[idx_v.at[0].at[0]], gbuf.at[0],
                         semg.at[0])

        def step(j, carry):
            sl = lax.rem(j, NSLOT)
            sl1 = lax.rem(j + 1, NSLOT)
            sl2 = lax.rem(j + 2, NSLOT)

            @pl.when(j + 2 < nchunks)
            def _():
                pltpu.async_copy(idx_hbm.at[s].at[j + 2], idx_v.at[sl2],
                                 semi.at[sl2])

            @pl.when(j + 1 < nchunks)
            def _():
                pltpu.make_async_copy(idx_hbm.at[s].at[j + 1], idx_v.at[sl1],
                                      semi.at[sl1]).wait()
                pltpu.async_copy(y_hbm.at[t].at[idx_v.at[sl1].at[0]],
                                 gbuf.at[sl1], semg.at[sl1])

            pltpu.make_async_copy(y_hbm.at[t].at[idx_v.at[sl].at[0]],
                                  gbuf.at[sl], semg.at[sl]).wait()
            pltpu.sync_copy(gbuf.at[sl], acc.at[idx_v.at[sl].at[1]], add=True)
            return carry

        lax.fori_loop(0, nchunks, step, 0)
        plsc.subcore_barrier()
        pltpu.sync_copy(acc.at[pl.ds(s * RPT, RPT)],
                        out_hbm.at[t].at[pl.ds(s * RPT, RPT)])
        if tl < T // NCORES - 1:
            pltpu.sync_copy(zeros_hbm, acc.at[pl.ds(s * RPT, RPT)])
            plsc.subcore_barrier()


def _sc_degree(idx_tiles, ones16, zeros16):
    call = pl.kernel(
        _sc_deg_body,
        out_type=jax.ShapeDtypeStruct((NCORES, NPAD, F), jnp.float32),
        mesh=_mesh(),
        scratch_types=[
            pltpu.VMEM((2, CHUNK), jnp.int32),
            pltpu.VMEM((CHUNK, F), jnp.float32),
            pltpu.VMEM_SHARED((NPAD, F), jnp.float32),
            pltpu.SemaphoreType.DMA,
        ],
    )
    return call(idx_tiles, ones16, zeros16)


def _sc_aggregate(y, idx_tiles, zerosf):
    call = pl.kernel(
        _sc_agg_body,
        out_type=jax.ShapeDtypeStruct((T, NPAD, F), jnp.float32),
        mesh=_mesh(),
        scratch_types=[
            pltpu.VMEM((NSLOT, 2, CHUNK), jnp.int32),
            pltpu.VMEM((NSLOT, CHUNK, F), jnp.float32),
            pltpu.VMEM_SHARED((NPAD, F), jnp.float32),
            pltpu.SemaphoreType.DMA((NSLOT,)),
            pltpu.SemaphoreType.DMA((NSLOT,)),
        ],
    )
    return call(y, idx_tiles, zerosf)


def _rsqrt_deg(d_ref):
    return lax.rsqrt(d_ref[0, :, 0:1] + d_ref[1, :, 0:1] + 1.0)


def _scale_body(g_ref, d_ref, o_ref):
    o_ref[0] = g_ref[0] * _rsqrt_deg(d_ref)


def _layer_body(acc_ref, y_ref, d_ref, w_ref, b_ref, o_ref):
    r = _rsqrt_deg(d_ref)
    sx = (acc_ref[0] + y_ref[0]) * r
    h = jnp.maximum(
        jnp.dot(sx, w_ref[...], preferred_element_type=jnp.float32,
                precision=_PREC) + b_ref[...][None, :], 0.0)
    o_ref[0] = h * r


def _tail_body(acc_ref, y_ref, d_ref, w2_ref, b2_ref, wd1_ref, bd1_ref,
               wd2_ref, bd2_ref, wi_ref, bl_ref, o_ref):
    r = _rsqrt_deg(d_ref)
    sx = (acc_ref[0] + y_ref[0]) * r
    h2 = jnp.maximum(
        jnp.dot(sx, w2_ref[...], preferred_element_type=jnp.float32,
                precision=_PREC) + b2_ref[...][None, :], 0.0)
    x3 = jnp.dot(h2, wd1_ref[...], preferred_element_type=jnp.float32,
                 precision=_PREC) + bd1_ref[...][None, :]
    x4 = jnp.dot(x3, wd2_ref[...], preferred_element_type=jnp.float32,
                 precision=_PREC) + bd2_ref[...][None, :]
    xw = jnp.dot(x4, wi_ref[...], preferred_element_type=jnp.float32,
                 precision=_PREC) + bl_ref[...][None, :]
    o_ref[0] = xw


def _lstm_body(x_ref, whp_ref, a_ref, o_ref):
    whp = whp_ref[...]
    a = a_ref[...]
    UNROLL = 8

    def one(x, h, c):
        z = x + jnp.dot(h, whp, preferred_element_type=jnp.float32)
        q = jnp.tanh(z * a)
        qs = q * 0.5 + 0.5
        g_al = jnp.roll(q, -2 * U, axis=1)
        f_al = jnp.roll(qs, -U, axis=1)
        o_al = jnp.roll(qs, -3 * U, axis=1)
        c2 = f_al * c + qs * g_al
        h2 = o_al * jnp.tanh(c2)
        return h2, c2

    def step(i, hc):
        h, c = hc
        xblk = x_ref[pl.ds(i * UNROLL, UNROLL)]
        for k in range(UNROLL):
            h, c = one(xblk[k], h, c)
        return (h, c)

    init = (jnp.zeros((T, 4 * U), jnp.float32),
            jnp.zeros((T, 4 * U), jnp.float32))
    h, _ = lax.fori_loop(0, x_ref.shape[0] // UNROLL, step, init)
    o_ref[...] = h[:, 0:U]


def _final_body(flat_ref, w_ref, b_ref, o_ref):
    o_ref[...] = jnp.dot(flat_ref[...], w_ref[...],
                         preferred_element_type=jnp.float32,
                         precision=_PREC) + b_ref[...]


def _tc_scale(graphs, degp):
    return pl.pallas_call(
        _scale_body,
        grid=(T, GRID_N),
        in_specs=[
            pl.BlockSpec((1, NB, F), lambda t, n: (t, n, 0)),
            pl.BlockSpec((NCORES, NB, 128), lambda t, n: (0, n, 0)),
        ],
        out_specs=pl.BlockSpec((1, NB, F), lambda t, n: (t, n, 0)),
        out_shape=jax.ShapeDtypeStruct((T, N, F), jnp.float32),
    )(graphs, degp)


def _tc_layer(acc, y, degp, w, b):
    return pl.pallas_call(
        _layer_body,
        grid=(T, GRID_N),
        in_specs=[
            pl.BlockSpec((1, NB, F), lambda t, n: (t, n, 0)),
            pl.BlockSpec((1, NB, F), lambda t, n: (t, n, 0)),
            pl.BlockSpec((NCORES, NB, 128), lambda t, n: (0, n, 0)),
            pl.BlockSpec((F, F), lambda t, n: (0, 0)),
            pl.BlockSpec((F,), lambda t, n: (0,)),
        ],
        out_specs=pl.BlockSpec((1, NB, F), lambda t, n: (t, n, 0)),
        out_shape=jax.ShapeDtypeStruct((T, N, F), jnp.float32),
    )(acc, y, degp, w, b)


def _tc_tail(acc, y, degp, W2, b2, Wd1, bd1, Wd2, bd2, Wi, bl):
    wspec = pl.BlockSpec((F, F), lambda t, n: (0, 0))
    bspec = pl.BlockSpec((F,), lambda t, n: (0,))
    return pl.pallas_call(
        _tail_body,
        grid=(T, GRID_N),
        in_specs=[
            pl.BlockSpec((1, NB, F), lambda t, n: (t, n, 0)),
            pl.BlockSpec((1, NB, F), lambda t, n: (t, n, 0)),
            pl.BlockSpec((NCORES, NB, 128), lambda t, n: (0, n, 0)),
            wspec, bspec, wspec, bspec, wspec, bspec,
            pl.BlockSpec((F, 4 * U), lambda t, n: (0, 0)),
            pl.BlockSpec((4 * U,), lambda t, n: (0,)),
        ],
        out_specs=pl.BlockSpec((1, NB, 4 * U), lambda t, n: (t, n, 0)),
        out_shape=jax.ShapeDtypeStruct((T, N, 4 * U), jnp.float32),
    )(acc, y, degp, W2, b2, Wd1, bd1, Wd2, bd2, Wi, bl)


def _tc_lstm(xw, Wh):
    whp = jnp.zeros((4 * U, 4 * U), jnp.float32).at[0:U, :].set(Wh)
    a = jnp.concatenate([
        jnp.full((1, 2 * U), 0.5, jnp.float32),
        jnp.full((1, U), 1.0, jnp.float32),
        jnp.full((1, U), 0.5, jnp.float32)], axis=1)
    return pl.pallas_call(
        _lstm_body,
        out_shape=jax.ShapeDtypeStruct((T, U), jnp.float32),
    )(xw, whp, a)


def _tc_final(flat, Wd3, bd3):
    CB = 6400
    return pl.pallas_call(
        _final_body,
        grid=(N * U // CB,),
        in_specs=[
            pl.BlockSpec((1, T * U), lambda n: (0, 0)),
            pl.BlockSpec((T * U, CB), lambda n: (0, n)),
            pl.BlockSpec((1, CB), lambda n: (0, n)),
        ],
        out_specs=pl.BlockSpec((1, CB), lambda n: (0, n)),
        out_shape=jax.ShapeDtypeStruct((1, N * U), jnp.float32),
    )(flat, Wd3, bd3)


def kernel(graphs, edge_index, edge_attr, W1, b1, W2, b2, Wd1, bd1, Wd2, bd2,
           Wi, Wh, bl, Wd3, bd3):
    src = edge_index[0]
    dst = edge_index[1]
    E = src.shape[0]
    per_tile = -(-(E // NSUB) // CHUNK) * CHUNK
    tot = per_tile * NSUB
    pad = tot - E
    src_tiles = jnp.concatenate(
        [src, jnp.zeros((pad,), jnp.int32)]).reshape(NSUB, -1, CHUNK)
    dst_tiles = jnp.concatenate(
        [dst, jnp.full((pad,), SCRAP, jnp.int32)]).reshape(NSUB, -1, CHUNK)
    idx_tiles = jnp.stack([src_tiles, dst_tiles], axis=2)
    ones16 = jnp.ones((CHUNK, F), jnp.float32)
    zeros16 = jnp.zeros((RPT, F), jnp.float32)
    zerosf = jnp.zeros((RPT, F), jnp.float32)

    degp = _sc_degree(idx_tiles, ones16, zeros16)
    y0 = _tc_scale(graphs, degp)
    acc1 = _sc_aggregate(y0, idx_tiles, zerosf)
    y1 = _tc_layer(acc1, y0, degp, W1, b1)
    acc2 = _sc_aggregate(y1, idx_tiles, zerosf)
    xw = _tc_tail(acc2, y1, degp, W2, b2, Wd1, bd1, Wd2, bd2, Wi, bl)
    h_last = _tc_lstm(jnp.swapaxes(xw, 0, 1), Wh)
    out = _tc_final(h_last.reshape(1, T * U), Wd3, bd3.reshape(1, N * U))
    return out.reshape(N, U)


# speedup vs baseline: 2.0420x; 1.0316x over previous
import functools

import jax
import jax.numpy as jnp
from jax import lax
from jax.experimental import pallas as pl
from jax.experimental.pallas import tpu as pltpu
from jax.experimental.pallas import tpu_sc as plsc

N = 10000
F = 128
T = 8
U = 16
NPAD = 10240
SCRAP = 10016
NSUB = 16
NCORES = 2
CHUNK = 120
NSLOT = 3
RPT = NPAD // NSUB
NB = 400
GRID_N = N // NB


@functools.lru_cache(maxsize=1)
def _mesh():
    return plsc.VectorSubcoreMesh(
        core_axis_name="c", subcore_axis_name="s",
        num_cores=NCORES, num_subcores=NSUB)


_PREC = lax.Precision.HIGHEST


def _sc_deg_body(dst_hbm, ones_hbm, zeros_hbm, out_hbm, idx_v, ones_v, acc, sem):
    c = lax.axis_index("c")
    s = lax.axis_index("s")
    nchunks = dst_hbm.shape[1]
    half = (nchunks + 1) // 2
    pltpu.sync_copy(ones_hbm, ones_v)
    pltpu.sync_copy(zeros_hbm, acc.at[pl.ds(s * RPT, RPT)])
    plsc.subcore_barrier()

    def step(j, carry):
        pltpu.sync_copy(dst_hbm.at[s].at[j], idx_v)
        pltpu.sync_copy(ones_v, acc.at[idx_v.at[1]], add=True)
        return carry

    lax.fori_loop(c * half, half + (nchunks - half) * c, step, 0)
    plsc.subcore_barrier()
    pltpu.sync_copy(acc.at[pl.ds(s * RPT, RPT)],
                    out_hbm.at[c].at[pl.ds(s * RPT, RPT)])


def _sc_agg_body(y_hbm, idx_hbm, zeros_hbm, out_hbm,
                 idx_v, gbuf, acc, semi, semg):
    c = lax.axis_index("c")
    s = lax.axis_index("s")
    nchunks = idx_hbm.shape[1]
    pltpu.sync_copy(zeros_hbm, acc.at[pl.ds(s * RPT, RPT)])
    plsc.subcore_barrier()
    for tl in range(T // NCORES):
        t = c * (T // NCORES) + tl
        pltpu.async_copy(idx_hbm.at[s].at[0], idx_v.at[0], semi.at[0])
        pltpu.async_copy(idx_hbm.at[s].at[1], idx_v.at[1], semi.at[1])
        pltpu.make_async_copy(idx_hbm.at[s].at[0], idx_v.at[0],
                              semi.at[0]).wait()
        pltpu.async_copy(y_hbm.at[t].at[idx_v.at[0].at[0]], gbuf.at[0],
                         semg.at[0])

        def step(j, carry):
            sl = lax.rem(j, NSLOT)
            sl1 = lax.rem(j + 1, NSLOT)
            sl2 = lax.rem(j + 2, NSLOT)

            @pl.when(j + 2 < nchunks)
            def _():
                pltpu.async_copy(idx_hbm.at[s].at[j + 2], idx_v.at[sl2],
                                 semi.at[sl2])

            @pl.when(j + 1 < nchunks)
            def _():
                pltpu.make_async_copy(idx_hbm.at[s].at[j + 1], idx_v.at[sl1],
                                      semi.at[sl1]).wait()
                pltpu.async_copy(y_hbm.at[t].at[idx_v.at[sl1].at[0]],
                                 gbuf.at[sl1], semg.at[sl1])

            pltpu.make_async_copy(y_hbm.at[t].at[idx_v.at[sl].at[0]],
                                  gbuf.at[sl], semg.at[sl]).wait()
            pltpu.sync_copy(gbuf.at[sl], acc.at[idx_v.at[sl].at[1]], add=True)
            return carry

        lax.fori_loop(0, nchunks, step, 0)
        plsc.subcore_barrier()
        pltpu.sync_copy(acc.at[pl.ds(s * RPT, RPT)],
                        out_hbm.at[t].at[pl.ds(s * RPT, RPT)])
        if tl < T // NCORES - 1:
            pltpu.sync_copy(zeros_hbm, acc.at[pl.ds(s * RPT, RPT)])
            plsc.subcore_barrier()


def _sc_degree(idx_tiles, ones16, zeros16):
    call = pl.kernel(
        _sc_deg_body,
        out_type=jax.ShapeDtypeStruct((NCORES, NPAD, F), jnp.float32),
        mesh=_mesh(),
        scratch_types=[
            pltpu.VMEM((2, CHUNK), jnp.int32),
            pltpu.VMEM((CHUNK, F), jnp.float32),
            pltpu.VMEM_SHARED((NPAD, F), jnp.float32),
            pltpu.SemaphoreType.DMA,
        ],
    )
    return call(idx_tiles, ones16, zeros16)


def _sc_aggregate(y, idx_tiles, zerosf):
    call = pl.kernel(
        _sc_agg_body,
        out_type=jax.ShapeDtypeStruct((T, NPAD, F), jnp.float32),
        mesh=_mesh(),
        scratch_types=[
            pltpu.VMEM((NSLOT, 2, CHUNK), jnp.int32),
            pltpu.VMEM((NSLOT, CHUNK, F), jnp.float32),
            pltpu.VMEM_SHARED((NPAD, F), jnp.float32),
            pltpu.SemaphoreType.DMA((NSLOT,)),
            pltpu.SemaphoreType.DMA((NSLOT,)),
        ],
    )
    return call(y, idx_tiles, zerosf)


def _rsqrt_deg(d_ref):
    return lax.rsqrt(d_ref[0, :, 0:1] + d_ref[1, :, 0:1] + 1.0)


def _scale_body(g_ref, d_ref, o_ref):
    o_ref[0] = g_ref[0] * _rsqrt_deg(d_ref)


def _layer_body(acc_ref, y_ref, d_ref, w_ref, b_ref, o_ref):
    r = _rsqrt_deg(d_ref)
    sx = (acc_ref[0] + y_ref[0]) * r
    h = jnp.maximum(
        jnp.dot(sx, w_ref[...], preferred_element_type=jnp.float32,
                precision=_PREC) + b_ref[...][None, :], 0.0)
    o_ref[0] = h * r


def _tail_body(acc_ref, y_ref, d_ref, w2_ref, b2_ref, wd1_ref, bd1_ref,
               wd2_ref, bd2_ref, wi_ref, bl_ref, o_ref):
    r = _rsqrt_deg(d_ref)
    sx = (acc_ref[0] + y_ref[0]) * r
    h2 = jnp.maximum(
        jnp.dot(sx, w2_ref[...], preferred_element_type=jnp.float32,
                precision=_PREC) + b2_ref[...][None, :], 0.0)
    x3 = jnp.dot(h2, wd1_ref[...], preferred_element_type=jnp.float32,
                 precision=_PREC) + bd1_ref[...][None, :]
    x4 = jnp.dot(x3, wd2_ref[...], preferred_element_type=jnp.float32,
                 precision=_PREC) + bd2_ref[...][None, :]
    xw = jnp.dot(x4, wi_ref[...], preferred_element_type=jnp.float32,
                 precision=_PREC) + bl_ref[...][None, :]
    o_ref[0] = xw


def _lstm_body(x_ref, wh_ref, a_ref, o_ref):
    a = a_ref[...]
    UNROLL = 8
    whrows = [jnp.broadcast_to(wh_ref[k:k + 1, :], (T, 4 * U))
              for k in range(U)]

    def one(x, h, c):
        terms = [h[:, k:k + 1] * whrows[k] for k in range(U)]
        while len(terms) > 1:
            nxt = [terms[i] + terms[i + 1] for i in range(0, len(terms) - 1, 2)]
            if len(terms) % 2:
                nxt.append(terms[-1])
            terms = nxt
        z = x + terms[0]
        q = jnp.tanh(z * a)
        qs = q * 0.5 + 0.5
        g_al = jnp.roll(q, -2 * U, axis=1)
        f_al = jnp.roll(qs, -U, axis=1)
        o_al = jnp.roll(qs, -3 * U, axis=1)
        c2 = f_al * c + qs * g_al
        h2 = o_al * jnp.tanh(c2)
        return h2, c2

    def step(i, hc):
        h, c = hc
        xblk = x_ref[pl.ds(i * UNROLL, UNROLL)]
        for k in range(UNROLL):
            h, c = one(xblk[k], h, c)
        return (h, c)

    init = (jnp.zeros((T, 4 * U), jnp.float32),
            jnp.zeros((T, 4 * U), jnp.float32))
    h, _ = lax.fori_loop(0, x_ref.shape[0] // UNROLL, step, init)
    o_ref[...] = h[:, 0:U]


def _final_body(flat_ref, w_ref, b_ref, o_ref):
    o_ref[...] = jnp.dot(flat_ref[...], w_ref[...],
                         preferred_element_type=jnp.float32,
                         precision=_PREC) + b_ref[...]


def _tc_scale(graphs, degp):
    return pl.pallas_call(
        _scale_body,
        grid=(T, GRID_N),
        in_specs=[
            pl.BlockSpec((1, NB, F), lambda t, n: (t, n, 0)),
            pl.BlockSpec((NCORES, NB, 128), lambda t, n: (0, n, 0)),
        ],
        out_specs=pl.BlockSpec((1, NB, F), lambda t, n: (t, n, 0)),
        out_shape=jax.ShapeDtypeStruct((T, N, F), jnp.float32),
    )(graphs, degp)


def _tc_layer(acc, y, degp, w, b):
    return pl.pallas_call(
        _layer_body,
        grid=(T, GRID_N),
        in_specs=[
            pl.BlockSpec((1, NB, F), lambda t, n: (t, n, 0)),
            pl.BlockSpec((1, NB, F), lambda t, n: (t, n, 0)),
            pl.BlockSpec((NCORES, NB, 128), lambda t, n: (0, n, 0)),
            pl.BlockSpec((F, F), lambda t, n: (0, 0)),
            pl.BlockSpec((F,), lambda t, n: (0,)),
        ],
        out_specs=pl.BlockSpec((1, NB, F), lambda t, n: (t, n, 0)),
        out_shape=jax.ShapeDtypeStruct((T, N, F), jnp.float32),
    )(acc, y, degp, w, b)


def _tc_tail(acc, y, degp, W2, b2, Wd1, bd1, Wd2, bd2, Wi, bl):
    wspec = pl.BlockSpec((F, F), lambda t, n: (0, 0))
    bspec = pl.BlockSpec((F,), lambda t, n: (0,))
    return pl.pallas_call(
        _tail_body,
        grid=(T, GRID_N),
        in_specs=[
            pl.BlockSpec((1, NB, F), lambda t, n: (t, n, 0)),
            pl.BlockSpec((1, NB, F), lambda t, n: (t, n, 0)),
            pl.BlockSpec((NCORES, NB, 128), lambda t, n: (0, n, 0)),
            wspec, bspec, wspec, bspec, wspec, bspec,
            pl.BlockSpec((F, 4 * U), lambda t, n: (0, 0)),
            pl.BlockSpec((4 * U,), lambda t, n: (0,)),
        ],
        out_specs=pl.BlockSpec((1, NB, 4 * U), lambda t, n: (t, n, 0)),
        out_shape=jax.ShapeDtypeStruct((T, N, 4 * U), jnp.float32),
    )(acc, y, degp, W2, b2, Wd1, bd1, Wd2, bd2, Wi, bl)


def _tc_lstm(xw, Wh):
    a = jnp.concatenate([
        jnp.full((1, 2 * U), 0.5, jnp.float32),
        jnp.full((1, U), 1.0, jnp.float32),
        jnp.full((1, U), 0.5, jnp.float32)], axis=1)
    return pl.pallas_call(
        _lstm_body,
        out_shape=jax.ShapeDtypeStruct((T, U), jnp.float32),
    )(xw, Wh, a)


def _tc_final(flat, Wd3, bd3):
    CB = 6400
    return pl.pallas_call(
        _final_body,
        grid=(N * U // CB,),
        in_specs=[
            pl.BlockSpec((1, T * U), lambda n: (0, 0)),
            pl.BlockSpec((T * U, CB), lambda n: (0, n)),
            pl.BlockSpec((1, CB), lambda n: (0, n)),
        ],
        out_specs=pl.BlockSpec((1, CB), lambda n: (0, n)),
        out_shape=jax.ShapeDtypeStruct((1, N * U), jnp.float32),
    )(flat, Wd3, bd3)


def kernel(graphs, edge_index, edge_attr, W1, b1, W2, b2, Wd1, bd1, Wd2, bd2,
           Wi, Wh, bl, Wd3, bd3):
    src = edge_index[0]
    dst = edge_index[1]
    E = src.shape[0]
    per_tile = -(-(E // NSUB) // CHUNK) * CHUNK
    tot = per_tile * NSUB
    pad = tot - E
    src_tiles = jnp.concatenate(
        [src, jnp.zeros((pad,), jnp.int32)]).reshape(NSUB, -1, CHUNK)
    dst_tiles = jnp.concatenate(
        [dst, jnp.full((pad,), SCRAP, jnp.int32)]).reshape(NSUB, -1, CHUNK)
    idx_tiles = jnp.stack([src_tiles, dst_tiles], axis=2)
    ones16 = jnp.ones((CHUNK, F), jnp.float32)
    zeros16 = jnp.zeros((RPT, F), jnp.float32)
    zerosf = jnp.zeros((RPT, F), jnp.float32)

    degp = _sc_degree(idx_tiles, ones16, zeros16)
    y0 = _tc_scale(graphs, degp)
    acc1 = _sc_aggregate(y0, idx_tiles, zerosf)
    y1 = _tc_layer(acc1, y0, degp, W1, b1)
    acc2 = _sc_aggregate(y1, idx_tiles, zerosf)
    xw = _tc_tail(acc2, y1, degp, W2, b2, Wd1, bd1, Wd2, bd2, Wi, bl)
    h_last = _tc_lstm(jnp.swapaxes(xw, 0, 1), Wh)
    out = _tc_final(h_last.reshape(1, T * U), Wd3, bd3.reshape(1, N * U))
    return out.reshape(N, U)


# speedup vs baseline: 2.0708x; 1.0141x over previous
import functools

import jax
import jax.numpy as jnp
from jax import lax
from jax.experimental import pallas as pl
from jax.experimental.pallas import tpu as pltpu
from jax.experimental.pallas import tpu_sc as plsc

N = 10000
F = 128
T = 8
U = 16
NPAD = 10240
SCRAP = 10016
NSUB = 16
NCORES = 2
CHUNK = 120
NSLOT = 3
RPT = NPAD // NSUB
NB = 400
GRID_N = N // NB


@functools.lru_cache(maxsize=1)
def _mesh():
    return plsc.VectorSubcoreMesh(
        core_axis_name="c", subcore_axis_name="s",
        num_cores=NCORES, num_subcores=NSUB)


_PREC = lax.Precision.HIGHEST


def _sc_deg_body(dst_hbm, ones_hbm, zeros_hbm, out_hbm, idx_v, ones_v, acc, sem):
    c = lax.axis_index("c")
    s = lax.axis_index("s")
    nchunks = dst_hbm.shape[1]
    half = (nchunks + 1) // 2
    pltpu.sync_copy(ones_hbm, ones_v)
    pltpu.sync_copy(zeros_hbm, acc.at[pl.ds(s * RPT, RPT)])
    plsc.subcore_barrier()

    def step(j, carry):
        pltpu.sync_copy(dst_hbm.at[s].at[j], idx_v)
        pltpu.sync_copy(ones_v, acc.at[idx_v.at[1]], add=True)
        return carry

    lax.fori_loop(c * half, half + (nchunks - half) * c, step, 0)
    plsc.subcore_barrier()
    pltpu.sync_copy(acc.at[pl.ds(s * RPT, RPT)],
                    out_hbm.at[c].at[pl.ds(s * RPT, RPT)])


def _sc_agg_body(y_hbm, idx_hbm, zeros_hbm, out_hbm,
                 idx_v, gbuf, acc, semi, semg):
    c = lax.axis_index("c")
    s = lax.axis_index("s")
    nchunks = idx_hbm.shape[1]
    pltpu.sync_copy(zeros_hbm, acc.at[pl.ds(s * RPT, RPT)])
    plsc.subcore_barrier()
    for tl in range(T // NCORES):
        t = c * (T // NCORES) + tl
        pltpu.async_copy(idx_hbm.at[s].at[0], idx_v.at[0], semi.at[0])
        pltpu.async_copy(idx_hbm.at[s].at[1], idx_v.at[1], semi.at[1])
        pltpu.make_async_copy(idx_hbm.at[s].at[0], idx_v.at[0],
                              semi.at[0]).wait()
        pltpu.async_copy(y_hbm.at[t].at[idx_v.at[0].at[0]], gbuf.at[0],
                         semg.at[0])

        def step(j, carry):
            sl = lax.rem(j, NSLOT)
            sl1 = lax.rem(j + 1, NSLOT)
            sl2 = lax.rem(j + 2, NSLOT)

            @pl.when(j + 2 < nchunks)
            def _():
                pltpu.async_copy(idx_hbm.at[s].at[j + 2], idx_v.at[sl2],
                                 semi.at[sl2])

            @pl.when(j + 1 < nchunks)
            def _():
                pltpu.make_async_copy(idx_hbm.at[s].at[j + 1], idx_v.at[sl1],
                                      semi.at[sl1]).wait()
                pltpu.async_copy(y_hbm.at[t].at[idx_v.at[sl1].at[0]],
                                 gbuf.at[sl1], semg.at[sl1])

            pltpu.make_async_copy(y_hbm.at[t].at[idx_v.at[sl].at[0]],
                                  gbuf.at[sl], semg.at[sl]).wait()
            pltpu.sync_copy(gbuf.at[sl], acc.at[idx_v.at[sl].at[1]], add=True)
            return carry

        lax.fori_loop(0, nchunks, step, 0)
        plsc.subcore_barrier()
        pltpu.sync_copy(acc.at[pl.ds(s * RPT, RPT)],
                        out_hbm.at[t].at[pl.ds(s * RPT, RPT)])
        if tl < T // NCORES - 1:
            pltpu.sync_copy(zeros_hbm, acc.at[pl.ds(s * RPT, RPT)])
            plsc.subcore_barrier()


def _sc_degree(idx_tiles, ones16, zeros16):
    call = pl.kernel(
        _sc_deg_body,
        out_type=jax.ShapeDtypeStruct((NCORES, NPAD, F), jnp.float32),
        mesh=_mesh(),
        scratch_types=[
            pltpu.VMEM((2, CHUNK), jnp.int32),
            pltpu.VMEM((CHUNK, F), jnp.float32),
            pltpu.VMEM_SHARED((NPAD, F), jnp.float32),
            pltpu.SemaphoreType.DMA,
        ],
    )
    return call(idx_tiles, ones16, zeros16)


def _sc_aggregate(y, idx_tiles, zerosf):
    call = pl.kernel(
        _sc_agg_body,
        out_type=jax.ShapeDtypeStruct((T, NPAD, F), jnp.float32),
        mesh=_mesh(),
        scratch_types=[
            pltpu.VMEM((NSLOT, 2, CHUNK), jnp.int32),
            pltpu.VMEM((NSLOT, CHUNK, F), jnp.float32),
            pltpu.VMEM_SHARED((NPAD, F), jnp.float32),
            pltpu.SemaphoreType.DMA((NSLOT,)),
            pltpu.SemaphoreType.DMA((NSLOT,)),
        ],
    )
    return call(y, idx_tiles, zerosf)


def _rsqrt_deg(d_ref):
    return lax.rsqrt(d_ref[0, :, 0:1] + d_ref[1, :, 0:1] + 1.0)


def _scale_body(g_ref, d_ref, o_ref):
    o_ref[0] = g_ref[0] * _rsqrt_deg(d_ref)


def _layer_body(acc_ref, y_ref, d_ref, w_ref, b_ref, o_ref):
    r = _rsqrt_deg(d_ref)
    sx = (acc_ref[0] + y_ref[0]) * r
    h = jnp.maximum(
        jnp.dot(sx, w_ref[...], preferred_element_type=jnp.float32,
                precision=_PREC) + b_ref[...][None, :], 0.0)
    o_ref[0] = h * r


def _tail_body(acc_ref, y_ref, d_ref, w2_ref, b2_ref, wd1_ref, bd1_ref,
               wd2_ref, bd2_ref, wi_ref, bl_ref, o_ref):
    r = _rsqrt_deg(d_ref)
    sx = (acc_ref[0] + y_ref[0]) * r
    h2 = jnp.maximum(
        jnp.dot(sx, w2_ref[...], preferred_element_type=jnp.float32,
                precision=_PREC) + b2_ref[...][None, :], 0.0)
    x3 = jnp.dot(h2, wd1_ref[...], preferred_element_type=jnp.float32,
                 precision=_PREC) + bd1_ref[...][None, :]
    x4 = jnp.dot(x3, wd2_ref[...], preferred_element_type=jnp.float32,
                 precision=_PREC) + bd2_ref[...][None, :]
    xw = jnp.dot(x4, wi_ref[...], preferred_element_type=jnp.float32,
                 precision=_PREC) + bl_ref[...][None, :]
    o_ref[0] = xw


def _lstm_body(x_ref, wh_ref, o_ref):
    UNROLL = 8
    whrows = [jnp.broadcast_to(wh_ref[k:k + 1, :], (T, 4 * U))
              for k in range(U)]

    def one(x, h, c):
        terms = [h[:, k:k + 1] * whrows[k] for k in range(U)]
        while len(terms) > 1:
            nxt = [terms[i] + terms[i + 1] for i in range(0, len(terms) - 1, 2)]
            if len(terms) % 2:
                nxt.append(terms[-1])
            terms = nxt
        z = x + terms[0]
        q = jnp.tanh(z)
        qs = q * 0.5 + 0.5
        g_al = jnp.roll(q, -2 * U, axis=1)
        f_al = jnp.roll(qs, -U, axis=1)
        o_al = jnp.roll(qs, -3 * U, axis=1)
        c2 = f_al * c + qs * g_al
        h2 = o_al * jnp.tanh(c2)
        return h2, c2

    def step(i, hc):
        h, c = hc
        xblk = x_ref[pl.ds(i * UNROLL, UNROLL)]
        for k in range(UNROLL):
            h, c = one(xblk[k], h, c)
        return (h, c)

    init = (jnp.zeros((T, 4 * U), jnp.float32),
            jnp.zeros((T, 4 * U), jnp.float32))
    h, _ = lax.fori_loop(0, x_ref.shape[0] // UNROLL, step, init)
    o_ref[...] = h[:, 0:U]


def _final_body(flat_ref, w_ref, b_ref, o_ref):
    o_ref[...] = jnp.dot(flat_ref[...], w_ref[...],
                         preferred_element_type=jnp.float32,
                         precision=_PREC) + b_ref[...]


def _tc_scale(graphs, degp):
    return pl.pallas_call(
        _scale_body,
        grid=(GRID_N, T),
        in_specs=[
            pl.BlockSpec((1, NB, F), lambda n, t: (t, n, 0)),
            pl.BlockSpec((NCORES, NB, 128), lambda n, t: (0, n, 0)),
        ],
        out_specs=pl.BlockSpec((1, NB, F), lambda n, t: (t, n, 0)),
        out_shape=jax.ShapeDtypeStruct((T, N, F), jnp.float32),
    )(graphs, degp)


def _tc_layer(acc, y, degp, w, b):
    return pl.pallas_call(
        _layer_body,
        grid=(GRID_N, T),
        in_specs=[
            pl.BlockSpec((1, NB, F), lambda n, t: (t, n, 0)),
            pl.BlockSpec((1, NB, F), lambda n, t: (t, n, 0)),
            pl.BlockSpec((NCORES, NB, 128), lambda n, t: (0, n, 0)),
            pl.BlockSpec((F, F), lambda n, t: (0, 0)),
            pl.BlockSpec((F,), lambda n, t: (0,)),
        ],
        out_specs=pl.BlockSpec((1, NB, F), lambda n, t: (t, n, 0)),
        out_shape=jax.ShapeDtypeStruct((T, N, F), jnp.float32),
    )(acc, y, degp, w, b)


def _tc_tail(acc, y, degp, W2, b2, Wd1, bd1, Wd2, bd2, Wi, bl):
    wspec = pl.BlockSpec((F, F), lambda n, t: (0, 0))
    bspec = pl.BlockSpec((F,), lambda n, t: (0,))
    return pl.pallas_call(
        _tail_body,
        grid=(GRID_N, T),
        in_specs=[
            pl.BlockSpec((1, NB, F), lambda n, t: (t, n, 0)),
            pl.BlockSpec((1, NB, F), lambda n, t: (t, n, 0)),
            pl.BlockSpec((NCORES, NB, 128), lambda n, t: (0, n, 0)),
            wspec, bspec, wspec, bspec, wspec, bspec,
            pl.BlockSpec((F, 4 * U), lambda n, t: (0, 0)),
            pl.BlockSpec((4 * U,), lambda n, t: (0,)),
        ],
        out_specs=pl.BlockSpec((1, NB, 4 * U), lambda n, t: (t, n, 0)),
        out_shape=jax.ShapeDtypeStruct((T, N, 4 * U), jnp.float32),
    )(acc, y, degp, W2, b2, Wd1, bd1, Wd2, bd2, Wi, bl)


def _tc_lstm(xw, Wh):
    sv = jnp.concatenate([
        jnp.full((1, 2 * U), 0.5, jnp.float32),
        jnp.ones((1, U), jnp.float32),
        jnp.full((1, U), 0.5, jnp.float32)], axis=1)
    return pl.pallas_call(
        _lstm_body,
        out_shape=jax.ShapeDtypeStruct((T, U), jnp.float32),
    )(xw, Wh * sv)


def _tc_final(flat, Wd3, bd3):
    CB = 6400
    return pl.pallas_call(
        _final_body,
        grid=(N * U // CB,),
        in_specs=[
            pl.BlockSpec((1, T * U), lambda n: (0, 0)),
            pl.BlockSpec((T * U, CB), lambda n: (0, n)),
            pl.BlockSpec((1, CB), lambda n: (0, n)),
        ],
        out_specs=pl.BlockSpec((1, CB), lambda n: (0, n)),
        out_shape=jax.ShapeDtypeStruct((1, N * U), jnp.float32),
    )(flat, Wd3, bd3)


def kernel(graphs, edge_index, edge_attr, W1, b1, W2, b2, Wd1, bd1, Wd2, bd2,
           Wi, Wh, bl, Wd3, bd3):
    src = edge_index[0]
    dst = edge_index[1]
    E = src.shape[0]
    per_tile = -(-(E // NSUB) // CHUNK) * CHUNK
    tot = per_tile * NSUB
    pad = tot - E
    src_tiles = jnp.concatenate(
        [src, jnp.zeros((pad,), jnp.int32)]).reshape(NSUB, -1, CHUNK)
    dst_tiles = jnp.concatenate(
        [dst, jnp.full((pad,), SCRAP, jnp.int32)]).reshape(NSUB, -1, CHUNK)
    idx_tiles = jnp.stack([src_tiles, dst_tiles], axis=2)
    ones16 = jnp.ones((CHUNK, F), jnp.float32)
    zeros16 = jnp.zeros((RPT, F), jnp.float32)
    zerosf = jnp.zeros((RPT, F), jnp.float32)

    sv = jnp.concatenate([
        jnp.full((2 * U,), 0.5, jnp.float32),
        jnp.ones((U,), jnp.float32),
        jnp.full((U,), 0.5, jnp.float32)])
    Wi_s = Wi * sv[None, :]
    bl_s = bl * sv

    degp = _sc_degree(idx_tiles, ones16, zeros16)
    y0 = _tc_scale(graphs, degp)
    acc1 = _sc_aggregate(y0, idx_tiles, zerosf)
    y1 = _tc_layer(acc1, y0, degp, W1, b1)
    acc2 = _sc_aggregate(y1, idx_tiles, zerosf)
    xw = _tc_tail(acc2, y1, degp, W2, b2, Wd1, bd1, Wd2, bd2, Wi_s, bl_s)
    h_last = _tc_lstm(jnp.swapaxes(xw, 0, 1), Wh)
    out = _tc_final(h_last.reshape(1, T * U), Wd3, bd3.reshape(1, N * U))
    return out.reshape(N, U)
